# baseline (speedup 1.0000x reference)
import jax
import jax.numpy as jnp
from jax.experimental import pallas as pl

N = 50000
R = 8
OUT = 64
EMB = 64
G = 128


def _rgcn(x, src, dst, etype, Wr, Wroot, b):
    h = jnp.einsum('nd,rde->rne', x, Wr)
    msg = h[etype, src]
    keyi = dst * R + etype
    cnt = jnp.zeros((N * R,), jnp.float32).at[keyi].add(1.0)
    norm = 1.0 / jnp.maximum(cnt[keyi], 1.0)
    agg = jnp.zeros((N, Wr.shape[2]), jnp.float32).at[dst].add(msg * norm[:, None])
    return agg + x @ Wroot + b


def _bn(x, g, b):
    m = jnp.mean(x, axis=0)
    v = jnp.var(x, axis=0)
    return g * (x - m) / jnp.sqrt(v + 1e-5) + b


def _cls_kernel(p_ref, w_ref, b_ref, o_ref):
    o_ref[...] = p_ref[...] @ w_ref[...] + b_ref[...]


def kernel(x, edge_index, edge_type, batch, compress_W, compress_b, Wr1, Wroot1, b1, g1, be1, Wr2, Wroot2, b2, g2, be2, Wr3, Wroot3, b3, g3, be3, Wc, bc):
    src, dst = edge_index[0], edge_index[1]
    emb = x[:, :EMB] @ compress_W + compress_b
    h = jnp.concatenate([emb, x[:, EMB:]], axis=-1)
    h = jax.nn.relu(_bn(_rgcn(h, src, dst, edge_type, Wr1, Wroot1, b1), g1, be1))
    h = jax.nn.relu(_bn(_rgcn(h, src, dst, edge_type, Wr2, Wroot2, b2), g2, be2))
    h = jax.nn.relu(_bn(_rgcn(h, src, dst, edge_type, Wr3, Wroot3, b3), g3, be3))
    cnt = jnp.zeros((G,), jnp.float32).at[batch].add(1.0)
    s = jnp.zeros((G, OUT), jnp.float32).at[batch].add(h)
    x_mean = s / jnp.maximum(cnt, 1.0)[:, None]
    x_max = jax.ops.segment_max(h, batch, num_segments=G)
    x_max = jnp.where(cnt[:, None] > 0, x_max, 0.0)
    pooled = jnp.concatenate([x_mean, x_max], axis=-1)
    Wc_pad = jnp.pad(Wc, ((0, 0), (0, 125)))
    bc_pad = jnp.pad(bc, ((0, 125)))
    out = pl.pallas_call(
        _cls_kernel,
        out_shape=jax.ShapeDtypeStruct((G, 128), jnp.float32),
    )(pooled, Wc_pad, bc_pad[None, :])
    return out[:, :3]


# speedup vs baseline: 5.0160x; 5.0160x over previous
import functools

import jax
import jax.numpy as jnp
from jax import lax
from jax.experimental import pallas as pl
from jax.experimental.pallas import tpu as pltpu
from jax.experimental.pallas import tpu_sc as plsc

N = 50000
E = 800000
R = 8
OUT = 64
EMB = 64
COMP = 32
FEAT = 128
IN1 = 96
G = 128

NC = 2
NS = 16

EP = 819200
EROWS = EP // 128
TROWS = EROWS // NS
HROWS = TROWS // 2
NPAIR = HROWS // 2
NKEY = 409600
SROWS = 40
NSLAB = TROWS // SROWS
SPAIR = SROWS // 2
NTAB = R * N
NAGG = 50048
NAGT = NAGG // NS
NB = 25
BLK = 2000

_MESH = plsc.VectorSubcoreMesh(
    core_axis_name="c", subcore_axis_name="s", num_cores=NC, num_subcores=NS)
_SC_PARAMS = pltpu.CompilerParams(use_tc_tiling_on_sc=False)


def _pre_body(x_ref, w_ref, b_ref, o_ref):
    xb = x_ref[...]
    emb = xb[:, :EMB] @ w_ref[...] + b_ref[...]
    o_ref[...] = jnp.concatenate([emb, xb[:, EMB:]], axis=1)


def _pre(x, w, b):
    return pl.pallas_call(
        _pre_body,
        grid=(NB,),
        in_specs=[
            pl.BlockSpec((BLK, FEAT), lambda n: (n, 0)),
            pl.BlockSpec((EMB, COMP), lambda n: (0, 0)),
            pl.BlockSpec((1, COMP), lambda n: (0, 0)),
        ],
        out_specs=pl.BlockSpec((BLK, IN1), lambda n: (n, 0)),
        out_shape=jax.ShapeDtypeStruct((N, IN1), jnp.float32),
    )(x, w, b)


def _eprep_body(src_ref, dst_ref, et_ref, gidx_ref, key_ref):
    src = src_ref[...]
    dst = dst_ref[...]
    et = et_ref[...]
    gidx_ref[...] = et * N + src
    key_ref[...] = dst * R + et


def _eprep(src2, dst2, et2):
    return pl.pallas_call(
        _eprep_body,
        out_shape=[
            jax.ShapeDtypeStruct((EROWS, 128), jnp.int32),
            jax.ShapeDtypeStruct((EROWS, 128), jnp.int32),
        ],
    )(src2, dst2, et2)


def _einsum_body(x_ref, wr_ref, ha_ref, hb_ref):
    h = x_ref[...] @ wr_ref[0]
    ha_ref[...] = h[:, :32]
    hb_ref[...] = h[:, 32:]


def _einsum(x, wr, d):
    return pl.pallas_call(
        _einsum_body,
        grid=(R, NB),
        in_specs=[
            pl.BlockSpec((BLK, d), lambda r, n: (n, 0)),
            pl.BlockSpec((1, d, OUT), lambda r, n: (r, 0, 0)),
        ],
        out_specs=[
            pl.BlockSpec((BLK, 32), lambda r, n: (r * NB + n, 0)),
            pl.BlockSpec((BLK, 32), lambda r, n: (r * NB + n, 0)),
        ],
        out_shape=[
            jax.ShapeDtypeStruct((NTAB, 32), jnp.float32),
            jax.ShapeDtypeStruct((NTAB, 32), jnp.float32),
        ],
    )(x, wr)


def _bnsum_body(agga_ref, aggb_ref, x_ref, wr_ref, b_ref, z_ref, st_ref):
    agg = jnp.concatenate([agga_ref[...], aggb_ref[...]], axis=1)
    z = agg + x_ref[...] @ wr_ref[...] + b_ref[...]
    z_ref[...] = z
    s = jnp.sum(z, axis=0, keepdims=True)
    s2 = jnp.sum(z * z, axis=0, keepdims=True)
    st = jnp.concatenate([s, s2], axis=0)

    @pl.when(pl.program_id(0) == 0)
    def _():
        st_ref[...] = st

    @pl.when(pl.program_id(0) != 0)
    def _():
        st_ref[...] += st


def _bnsum(agga, aggb, x, wroot, b, d):
    return pl.pallas_call(
        _bnsum_body,
        grid=(NB,),
        in_specs=[
            pl.BlockSpec((BLK, 32), lambda n: (n, 0)),
            pl.BlockSpec((BLK, 32), lambda n: (n, 0)),
            pl.BlockSpec((BLK, d), lambda n: (n, 0)),
            pl.BlockSpec((d, OUT), lambda n: (0, 0)),
            pl.BlockSpec((1, OUT), lambda n: (0, 0)),
        ],
        out_specs=[
            pl.BlockSpec((BLK, OUT), lambda n: (n, 0)),
            pl.BlockSpec((2, OUT), lambda n: (0, 0)),
        ],
        out_shape=[
            jax.ShapeDtypeStruct((N, OUT), jnp.float32),
            jax.ShapeDtypeStruct((2, OUT), jnp.float32),
        ],
    )(agga, aggb, x, wroot, b)


def _bnapply_body(z_ref, st_ref, g_ref, be_ref, o_ref):
    m = st_ref[0:1, :] / N
    v = st_ref[1:2, :] / N - m * m
    zn = g_ref[...] * (z_ref[...] - m) / jnp.sqrt(v + 1e-5) + be_ref[...]
    o_ref[...] = jnp.maximum(zn, 0.0)


def _bnapply(z, st, g, be):
    return pl.pallas_call(
        _bnapply_body,
        grid=(NB,),
        in_specs=[
            pl.BlockSpec((BLK, OUT), lambda n: (n, 0)),
            pl.BlockSpec((2, OUT), lambda n: (0, 0)),
            pl.BlockSpec((1, OUT), lambda n: (0, 0)),
            pl.BlockSpec((1, OUT), lambda n: (0, 0)),
        ],
        out_specs=pl.BlockSpec((BLK, OUT), lambda n: (n, 0)),
        out_shape=jax.ShapeDtypeStruct((N, OUT), jnp.float32),
    )(z, st, g, be)


def _pool_body(h_ref, b_ref, wc_ref, bc_ref, o_ref, ssum, smax, scnt):
    n = pl.program_id(0)

    @pl.when(n == 0)
    def _():
        ssum[...] = jnp.zeros((G, OUT), jnp.float32)
        smax[...] = jnp.full((G, OUT), -1e30, jnp.float32)
        scnt[...] = jnp.zeros((G, 8), jnp.float32)

    hb = h_ref[...]
    bb = b_ref[...]
    lo = b_ref[0, 0]
    hi = b_ref[BLK - 1, 0]

    def body(g, _):
        mask = bb == g
        s_g = jnp.sum(jnp.where(mask, hb, 0.0), axis=0, keepdims=True)
        m_g = jnp.max(jnp.where(mask, hb, -1e30), axis=0, keepdims=True)
        c_g = jnp.sum(mask.astype(jnp.float32))
        ssum[pl.ds(g, 1), :] += s_g
        smax[pl.ds(g, 1), :] = jnp.maximum(smax[pl.ds(g, 1), :], m_g)
        scnt[pl.ds(g, 1), :] += c_g
        return 0

    lax.fori_loop(lo, hi + 1, body, 0)

    @pl.when(n == NB - 1)
    def _():
        cnt = scnt[:, 0:1]
        mean = ssum[...] / jnp.maximum(cnt, 1.0)
        mx = jnp.where(cnt > 0, smax[...], 0.0)
        pooled = jnp.concatenate([mean, mx], axis=1)
        o_ref[...] = pooled @ wc_ref[...] + bc_ref[...]


def _pool(h, batch3, wcp, bcp):
    return pl.pallas_call(
        _pool_body,
        grid=(NB,),
        in_specs=[
            pl.BlockSpec((BLK, OUT), lambda n: (n, 0)),
            pl.BlockSpec((BLK, 1), lambda n: (n, 0)),
            pl.BlockSpec((2 * OUT, 128), lambda n: (0, 0)),
            pl.BlockSpec((1, 128), lambda n: (0, 0)),
        ],
        out_specs=pl.BlockSpec((G, 128), lambda n: (0, 0)),
        out_shape=jax.ShapeDtypeStruct((G, 128), jnp.float32),
        scratch_shapes=[
            pltpu.VMEM((G, OUT), jnp.float32),
            pltpu.VMEM((G, OUT), jnp.float32),
            pltpu.VMEM((G, 8), jnp.float32),
        ],
    )(h, batch3, wcp, bcp)


@functools.partial(
    pl.kernel,
    out_type=jax.ShapeDtypeStruct((EROWS, 128), jnp.float32),
    mesh=_MESH,
    compiler_params=_SC_PARAMS,
    scratch_types=[
        pltpu.VMEM_SHARED((NKEY,), jnp.float32),
        pltpu.VMEM((TROWS, 128), jnp.int32),
        pltpu.VMEM((HROWS, 128), jnp.float32),
        pltpu.VMEM((128,), jnp.float32),
        pltpu.VMEM((128,), jnp.float32),
        pltpu.VMEM((1600,), jnp.float32),
    ],
)
def _scprep(key_hbm, norm_hbm, cnt_sh, kbuf, nbuf, ones_v, cval, zbuf):
    c = lax.axis_index("c")
    s = lax.axis_index("s")
    zero16 = jnp.zeros((16,), jnp.float32)
    one16 = jnp.ones((16,), jnp.float32)
    for i in range(100):
        zbuf[pl.ds(i * 16, 16)] = zero16
    for i in range(8):
        ones_v[pl.ds(i * 16, 16)] = one16
    for i in range(16):
        pltpu.sync_copy(zbuf, cnt_sh.at[pl.ds(s * 25600 + i * 1600, 1600)])
    plsc.subcore_barrier()
    pltpu.sync_copy(key_hbm.at[pl.ds(s * TROWS, TROWS)], kbuf)

    def count_chunk(i, _):
        pltpu.sync_copy(ones_v, cnt_sh.at[kbuf.at[i]], add=True)
        return 0

    lax.fori_loop(0, TROWS, count_chunk, 0)
    plsc.subcore_barrier()
    w = s * NC + c
    pltpu.sync_copy(key_hbm.at[pl.ds(w * HROWS, HROWS)],
                    kbuf.at[pl.ds(0, HROWS)])

    def norm_chunk(i, _):
        pltpu.sync_copy(cnt_sh.at[kbuf.at[i]], cval)
        for j in range(8):
            v = cval[pl.ds(j * 16, 16)]
            nbuf[i, pl.ds(j * 16, 16)] = 1.0 / jnp.maximum(v, 1.0)
        return 0

    lax.fori_loop(0, HROWS, norm_chunk, 0)
    pltpu.sync_copy(nbuf, norm_hbm.at[pl.ds(w * HROWS, HROWS)])


@functools.partial(
    pl.kernel,
    out_type=[
        jax.ShapeDtypeStruct((NAGG, 32), jnp.float32),
        jax.ShapeDtypeStruct((NAGG, 32), jnp.float32),
    ],
    mesh=_MESH,
    compiler_params=_SC_PARAMS,
    scratch_types=[
        pltpu.VMEM_SHARED((NAGG, 32), jnp.float32),
        pltpu.VMEM((SROWS, 128), jnp.int32),
        pltpu.VMEM((SROWS, 128), jnp.int32),
        pltpu.VMEM((SROWS, 128), jnp.float32),
        pltpu.VMEM((128, 32), jnp.float32),
        pltpu.VMEM((128, 32), jnp.float32),
        pltpu.SemaphoreType.DMA,
        pltpu.SemaphoreType.DMA,
        pltpu.SemaphoreType.DMA,
        pltpu.SemaphoreType.DMA,
    ],
)
def _scagg(ha_hbm, hb_hbm, gidx_hbm, dst_hbm, norm_hbm, outa_hbm, outb_hbm,
           agg_sh, gbuf, dbuf, nbuf, rows0, rows1,
           gsem0, gsem1, ssem0, ssem1):
    c = lax.axis_index("c")
    s = lax.axis_index("s")
    zero16 = jnp.zeros((16,), jnp.float32)
    for r in range(128):
        rows0[r, pl.ds(0, 16)] = zero16
        rows0[r, pl.ds(16, 16)] = zero16
    for i in range(24):
        pltpu.sync_copy(rows0, agg_sh.at[pl.ds(s * NAGT + i * 128, 128)])
    pltpu.sync_copy(rows0.at[pl.ds(0, 56)],
                    agg_sh.at[pl.ds(s * NAGT + 24 * 128, 56)])
    plsc.subcore_barrier()

    def scale(rows, i):
        for j in range(8):
            nv = nbuf[i, pl.ds(j * 16, 16)]
            for l in range(16):
                e = j * 16 + l
                sc = nv[l]
                rows[e, pl.ds(0, 16)] = rows[e, pl.ds(0, 16)] * sc
                rows[e, pl.ds(16, 16)] = rows[e, pl.ds(16, 16)] * sc

    def run(table):
        def slab(si, _):
            base = s * TROWS + si * SROWS
            pltpu.sync_copy(gidx_hbm.at[pl.ds(base, SROWS)], gbuf)
            pltpu.sync_copy(dst_hbm.at[pl.ds(base, SROWS)], dbuf)
            pltpu.sync_copy(norm_hbm.at[pl.ds(base, SROWS)], nbuf)
            pltpu.async_copy(table.at[gbuf.at[0]], rows0, gsem0)

            def pair(j, _):
                a = 2 * j
                b = a + 1
                pltpu.make_async_copy(
                    table.at[gbuf.at[a]], rows0, gsem0).wait()
                scale(rows0, a)

                @pl.when(j > 0)
                def _():
                    pltpu.make_async_copy(
                        rows1, agg_sh.at[dbuf.at[a]], ssem1).wait()

                pltpu.async_copy(table.at[gbuf.at[b]], rows1, gsem1)
                pltpu.async_copy(rows0, agg_sh.at[dbuf.at[a]], ssem0,
                                 add=True)
                pltpu.make_async_copy(
                    table.at[gbuf.at[b]], rows1, gsem1).wait()
                scale(rows1, b)
                pltpu.make_async_copy(
                    rows0, agg_sh.at[dbuf.at[a]], ssem0).wait()

                @pl.when(j < SPAIR - 1)
                def _():
                    pltpu.async_copy(table.at[gbuf.at[a + 2]], rows0, gsem0)

                pltpu.async_copy(rows1, agg_sh.at[dbuf.at[b]], ssem1,
                                 add=True)
                return 0

            lax.fori_loop(0, SPAIR, pair, 0)
            pltpu.make_async_copy(
                rows1, agg_sh.at[dbuf.at[SROWS - 1]], ssem1).wait()
            return 0

        lax.fori_loop(0, NSLAB, slab, 0)

    @pl.when(c == 0)
    def _():
        run(ha_hbm)

    @pl.when(c == 1)
    def _():
        run(hb_hbm)

    plsc.subcore_barrier()

    @pl.when(c == 0)
    def _():
        pltpu.sync_copy(agg_sh.at[pl.ds(s * NAGT, NAGT)],
                        outa_hbm.at[pl.ds(s * NAGT, NAGT)])

    @pl.when(c == 1)
    def _():
        pltpu.sync_copy(agg_sh.at[pl.ds(s * NAGT, NAGT)],
                        outb_hbm.at[pl.ds(s * NAGT, NAGT)])


def kernel(x, edge_index, edge_type, batch, compress_W, compress_b, Wr1, Wroot1, b1, g1, be1, Wr2, Wroot2, b2, g2, be2, Wr3, Wroot3, b3, g3, be3, Wc, bc):
    src, dst = edge_index[0], edge_index[1]
    pad = EP - E
    src2 = jnp.pad(src, (0, pad)).reshape(EROWS, 128)
    dst2 = jnp.pad(dst, (0, pad), constant_values=N).reshape(EROWS, 128)
    et2 = jnp.pad(edge_type, (0, pad)).reshape(EROWS, 128)
    gidx2, key2 = _eprep(src2, dst2, et2)
    norm2 = _scprep(key2)
    h = _pre(x, compress_W, compress_b[None, :])

    def layer(xin, d, Wr, Wroot, b, g, be):
        ha, hb = _einsum(xin, Wr, d)
        agga, aggb = _scagg(ha, hb, gidx2, dst2, norm2)
        z, st = _bnsum(agga, aggb, xin, Wroot, b[None, :], d)
        return _bnapply(z, st, g[None, :], be[None, :])

    h = layer(h, IN1, Wr1, Wroot1, b1, g1, be1)
    h = layer(h, OUT, Wr2, Wroot2, b2, g2, be2)
    h = layer(h, OUT, Wr3, Wroot3, b3, g3, be3)

    batch3 = batch.reshape(N, 1)
    wcp = jnp.pad(Wc, ((0, 0), (0, 125)))
    bcp = jnp.pad(bc, (0, 125))[None, :]
    out = _pool(h, batch3, wcp, bcp)
    return out[:, :3]


# speedup vs baseline: 5.0181x; 1.0004x over previous
import functools

import jax
import jax.numpy as jnp
from jax import lax
from jax.experimental import pallas as pl
from jax.experimental.pallas import tpu as pltpu
from jax.experimental.pallas import tpu_sc as plsc

N = 50000
E = 800000
R = 8
OUT = 64
EMB = 64
COMP = 32
FEAT = 128
IN1 = 96
G = 128

NC = 2
NS = 16

EP = 819200
EROWS = EP // 128
TROWS = EROWS // NS
HROWS = TROWS // 2
NPAIR = HROWS // 2
NKEY = 409600
SROWS = 40
NSLAB = TROWS // SROWS
SPAIR = SROWS // 2
NTAB = R * N
NAGG = 50048
NAGT = NAGG // NS
NB = 25
BLK = 2000

_MESH = plsc.VectorSubcoreMesh(
    core_axis_name="c", subcore_axis_name="s", num_cores=NC, num_subcores=NS)
_SC_PARAMS = pltpu.CompilerParams(use_tc_tiling_on_sc=False)


def _pre_body(x_ref, w_ref, b_ref, o_ref):
    xb = x_ref[...]
    emb = xb[:, :EMB] @ w_ref[...] + b_ref[...]
    o_ref[...] = jnp.concatenate([emb, xb[:, EMB:]], axis=1)


def _pre(x, w, b):
    return pl.pallas_call(
        _pre_body,
        grid=(NB,),
        in_specs=[
            pl.BlockSpec((BLK, FEAT), lambda n: (n, 0)),
            pl.BlockSpec((EMB, COMP), lambda n: (0, 0)),
            pl.BlockSpec((1, COMP), lambda n: (0, 0)),
        ],
        out_specs=pl.BlockSpec((BLK, IN1), lambda n: (n, 0)),
        out_shape=jax.ShapeDtypeStruct((N, IN1), jnp.float32),
    )(x, w, b)


def _eprep_body(src_ref, dst_ref, et_ref, gidx_ref, key_ref):
    src = src_ref[...]
    dst = dst_ref[...]
    et = et_ref[...]
    gidx_ref[...] = et * N + src
    key_ref[...] = dst * R + et


def _eprep(src2, dst2, et2):
    return pl.pallas_call(
        _eprep_body,
        out_shape=[
            jax.ShapeDtypeStruct((EROWS, 128), jnp.int32),
            jax.ShapeDtypeStruct((EROWS, 128), jnp.int32),
        ],
    )(src2, dst2, et2)


def _einsum_body(x_ref, wr_ref, ha_ref, hb_ref):
    h = x_ref[...] @ wr_ref[0]
    ha_ref[...] = h[:, :32]
    hb_ref[...] = h[:, 32:]


def _einsum(x, wr, d):
    return pl.pallas_call(
        _einsum_body,
        grid=(R, NB),
        in_specs=[
            pl.BlockSpec((BLK, d), lambda r, n: (n, 0)),
            pl.BlockSpec((1, d, OUT), lambda r, n: (r, 0, 0)),
        ],
        out_specs=[
            pl.BlockSpec((BLK, 32), lambda r, n: (r * NB + n, 0)),
            pl.BlockSpec((BLK, 32), lambda r, n: (r * NB + n, 0)),
        ],
        out_shape=[
            jax.ShapeDtypeStruct((NTAB, 32), jnp.float32),
            jax.ShapeDtypeStruct((NTAB, 32), jnp.float32),
        ],
    )(x, wr)


def _bnsum_body(agga_ref, aggb_ref, x_ref, wr_ref, b_ref, z_ref, st_ref):
    agg = jnp.concatenate([agga_ref[...], aggb_ref[...]], axis=1)
    z = agg + x_ref[...] @ wr_ref[...] + b_ref[...]
    z_ref[...] = z
    s = jnp.sum(z, axis=0, keepdims=True)
    s2 = jnp.sum(z * z, axis=0, keepdims=True)
    st = jnp.concatenate([s, s2], axis=0)

    @pl.when(pl.program_id(0) == 0)
    def _():
        st_ref[...] = st

    @pl.when(pl.program_id(0) != 0)
    def _():
        st_ref[...] += st


def _bnsum(agga, aggb, x, wroot, b, d):
    return pl.pallas_call(
        _bnsum_body,
        grid=(NB,),
        in_specs=[
            pl.BlockSpec((BLK, 32), lambda n: (n, 0)),
            pl.BlockSpec((BLK, 32), lambda n: (n, 0)),
            pl.BlockSpec((BLK, d), lambda n: (n, 0)),
            pl.BlockSpec((d, OUT), lambda n: (0, 0)),
            pl.BlockSpec((1, OUT), lambda n: (0, 0)),
        ],
        out_specs=[
            pl.BlockSpec((BLK, OUT), lambda n: (n, 0)),
            pl.BlockSpec((2, OUT), lambda n: (0, 0)),
        ],
        out_shape=[
            jax.ShapeDtypeStruct((N, OUT), jnp.float32),
            jax.ShapeDtypeStruct((2, OUT), jnp.float32),
        ],
    )(agga, aggb, x, wroot, b)


def _bnapply_body(z_ref, st_ref, g_ref, be_ref, o_ref):
    m = st_ref[0:1, :] / N
    v = st_ref[1:2, :] / N - m * m
    zn = g_ref[...] * (z_ref[...] - m) / jnp.sqrt(v + 1e-5) + be_ref[...]
    o_ref[...] = jnp.maximum(zn, 0.0)


def _bnapply(z, st, g, be):
    return pl.pallas_call(
        _bnapply_body,
        grid=(NB,),
        in_specs=[
            pl.BlockSpec((BLK, OUT), lambda n: (n, 0)),
            pl.BlockSpec((2, OUT), lambda n: (0, 0)),
            pl.BlockSpec((1, OUT), lambda n: (0, 0)),
            pl.BlockSpec((1, OUT), lambda n: (0, 0)),
        ],
        out_specs=pl.BlockSpec((BLK, OUT), lambda n: (n, 0)),
        out_shape=jax.ShapeDtypeStruct((N, OUT), jnp.float32),
    )(z, st, g, be)


def _pool_body(h_ref, b_ref, wc_ref, bc_ref, o_ref, ssum, smax, scnt):
    n = pl.program_id(0)

    @pl.when(n == 0)
    def _():
        ssum[...] = jnp.zeros((G, OUT), jnp.float32)
        smax[...] = jnp.full((G, OUT), -1e30, jnp.float32)
        scnt[...] = jnp.zeros((G, 8), jnp.float32)

    hb = h_ref[...]
    bb = b_ref[...]
    lo = b_ref[0, 0]
    hi = b_ref[BLK - 1, 0]

    def body(g, _):
        mask = bb == g
        s_g = jnp.sum(jnp.where(mask, hb, 0.0), axis=0, keepdims=True)
        m_g = jnp.max(jnp.where(mask, hb, -1e30), axis=0, keepdims=True)
        c_g = jnp.sum(mask.astype(jnp.float32))
        ssum[pl.ds(g, 1), :] += s_g
        smax[pl.ds(g, 1), :] = jnp.maximum(smax[pl.ds(g, 1), :], m_g)
        scnt[pl.ds(g, 1), :] += c_g
        return 0

    lax.fori_loop(lo, hi + 1, body, 0)

    @pl.when(n == NB - 1)
    def _():
        cnt = scnt[:, 0:1]
        mean = ssum[...] / jnp.maximum(cnt, 1.0)
        mx = jnp.where(cnt > 0, smax[...], 0.0)
        pooled = jnp.concatenate([mean, mx], axis=1)
        o_ref[...] = pooled @ wc_ref[...] + bc_ref[...]


def _pool(h, batch3, wcp, bcp):
    return pl.pallas_call(
        _pool_body,
        grid=(NB,),
        in_specs=[
            pl.BlockSpec((BLK, OUT), lambda n: (n, 0)),
            pl.BlockSpec((BLK, 1), lambda n: (n, 0)),
            pl.BlockSpec((2 * OUT, 128), lambda n: (0, 0)),
            pl.BlockSpec((1, 128), lambda n: (0, 0)),
        ],
        out_specs=pl.BlockSpec((G, 128), lambda n: (0, 0)),
        out_shape=jax.ShapeDtypeStruct((G, 128), jnp.float32),
        scratch_shapes=[
            pltpu.VMEM((G, OUT), jnp.float32),
            pltpu.VMEM((G, OUT), jnp.float32),
            pltpu.VMEM((G, 8), jnp.float32),
        ],
    )(h, batch3, wcp, bcp)


@functools.partial(
    pl.kernel,
    out_type=jax.ShapeDtypeStruct((EROWS, 128), jnp.float32),
    mesh=_MESH,
    compiler_params=_SC_PARAMS,
    scratch_types=[
        pltpu.VMEM_SHARED((NKEY,), jnp.float32),
        pltpu.VMEM((TROWS, 128), jnp.int32),
        pltpu.VMEM((HROWS, 128), jnp.float32),
        pltpu.VMEM((128,), jnp.float32),
        pltpu.VMEM((128,), jnp.float32),
        pltpu.VMEM((1600,), jnp.float32),
    ],
)
def _scprep(key_hbm, norm_hbm, cnt_sh, kbuf, nbuf, ones_v, cval, zbuf):
    c = lax.axis_index("c")
    s = lax.axis_index("s")
    zero16 = jnp.zeros((16,), jnp.float32)
    one16 = jnp.ones((16,), jnp.float32)
    for i in range(100):
        zbuf[pl.ds(i * 16, 16)] = zero16
    for i in range(8):
        ones_v[pl.ds(i * 16, 16)] = one16
    for i in range(16):
        pltpu.sync_copy(zbuf, cnt_sh.at[pl.ds(s * 25600 + i * 1600, 1600)])
    plsc.subcore_barrier()
    pltpu.sync_copy(key_hbm.at[pl.ds(s * TROWS, TROWS)], kbuf)

    def count_chunk(i, _):
        pltpu.sync_copy(ones_v, cnt_sh.at[kbuf.at[i]], add=True)
        return 0

    lax.fori_loop(0, TROWS, count_chunk, 0)
    plsc.subcore_barrier()
    w = s * NC + c
    pltpu.sync_copy(key_hbm.at[pl.ds(w * HROWS, HROWS)],
                    kbuf.at[pl.ds(0, HROWS)])

    def norm_chunk(i, _):
        pltpu.sync_copy(cnt_sh.at[kbuf.at[i]], cval)
        for j in range(8):
            v = cval[pl.ds(j * 16, 16)]
            nbuf[i, pl.ds(j * 16, 16)] = 1.0 / jnp.maximum(v, 1.0)
        return 0

    lax.fori_loop(0, HROWS, norm_chunk, 0)
    pltpu.sync_copy(nbuf, norm_hbm.at[pl.ds(w * HROWS, HROWS)])


@functools.partial(
    pl.kernel,
    out_type=[
        jax.ShapeDtypeStruct((NAGG, 32), jnp.float32),
        jax.ShapeDtypeStruct((NAGG, 32), jnp.float32),
    ],
    mesh=_MESH,
    compiler_params=_SC_PARAMS,
    scratch_types=[
        pltpu.VMEM_SHARED((NAGG, 32), jnp.float32),
        pltpu.VMEM((SROWS, 128), jnp.int32),
        pltpu.VMEM((SROWS, 128), jnp.int32),
        pltpu.VMEM((SROWS, 128), jnp.float32),
        pltpu.VMEM((128, 32), jnp.float32),
        pltpu.VMEM((128, 32), jnp.float32),
        pltpu.SemaphoreType.DMA,
        pltpu.SemaphoreType.DMA,
        pltpu.SemaphoreType.DMA,
        pltpu.SemaphoreType.DMA,
    ],
)
def _scagg(ha_hbm, hb_hbm, gidx_hbm, dst_hbm, norm_hbm, outa_hbm, outb_hbm,
           agg_sh, gbuf, dbuf, nbuf, rows0, rows1,
           gsem0, gsem1, ssem0, ssem1):
    c = lax.axis_index("c")
    s = lax.axis_index("s")
    zero16 = jnp.zeros((16,), jnp.float32)
    for r in range(128):
        rows0[r, pl.ds(0, 16)] = zero16
        rows0[r, pl.ds(16, 16)] = zero16
    for i in range(24):
        pltpu.sync_copy(rows0, agg_sh.at[pl.ds(s * NAGT + i * 128, 128)])
    pltpu.sync_copy(rows0.at[pl.ds(0, 56)],
                    agg_sh.at[pl.ds(s * NAGT + 24 * 128, 56)])
    plsc.subcore_barrier()

    splats = [jnp.full((16,), l, jnp.int32) for l in range(16)]

    def scale(rows, i):
        for j in range(8):
            nv = nbuf[i, pl.ds(j * 16, 16)]
            for l in range(16):
                e = j * 16 + l
                sc = nv.at[splats[l]].get(mode='promise_in_bounds')
                rows[e, pl.ds(0, 16)] = rows[e, pl.ds(0, 16)] * sc
                rows[e, pl.ds(16, 16)] = rows[e, pl.ds(16, 16)] * sc

    def run(table):
        def slab(si, _):
            base = s * TROWS + si * SROWS
            pltpu.sync_copy(gidx_hbm.at[pl.ds(base, SROWS)], gbuf)
            pltpu.sync_copy(dst_hbm.at[pl.ds(base, SROWS)], dbuf)
            pltpu.sync_copy(norm_hbm.at[pl.ds(base, SROWS)], nbuf)
            pltpu.async_copy(table.at[gbuf.at[0]], rows0, gsem0)

            def pair(j, _):
                a = 2 * j
                b = a + 1
                pltpu.make_async_copy(
                    table.at[gbuf.at[a]], rows0, gsem0).wait()
                scale(rows0, a)

                @pl.when(j > 0)
                def _():
                    pltpu.make_async_copy(
                        rows1, agg_sh.at[dbuf.at[a]], ssem1).wait()

                pltpu.async_copy(table.at[gbuf.at[b]], rows1, gsem1)
                pltpu.async_copy(rows0, agg_sh.at[dbuf.at[a]], ssem0,
                                 add=True)
                pltpu.make_async_copy(
                    table.at[gbuf.at[b]], rows1, gsem1).wait()
                scale(rows1, b)
                pltpu.make_async_copy(
                    rows0, agg_sh.at[dbuf.at[a]], ssem0).wait()

                @pl.when(j < SPAIR - 1)
                def _():
                    pltpu.async_copy(table.at[gbuf.at[a + 2]], rows0, gsem0)

                pltpu.async_copy(rows1, agg_sh.at[dbuf.at[b]], ssem1,
                                 add=True)
                return 0

            lax.fori_loop(0, SPAIR, pair, 0)
            pltpu.make_async_copy(
                rows1, agg_sh.at[dbuf.at[SROWS - 1]], ssem1).wait()
            return 0

        lax.fori_loop(0, NSLAB, slab, 0)

    @pl.when(c == 0)
    def _():
        run(ha_hbm)

    @pl.when(c == 1)
    def _():
        run(hb_hbm)

    plsc.subcore_barrier()

    @pl.when(c == 0)
    def _():
        pltpu.sync_copy(agg_sh.at[pl.ds(s * NAGT, NAGT)],
                        outa_hbm.at[pl.ds(s * NAGT, NAGT)])

    @pl.when(c == 1)
    def _():
        pltpu.sync_copy(agg_sh.at[pl.ds(s * NAGT, NAGT)],
                        outb_hbm.at[pl.ds(s * NAGT, NAGT)])


def kernel(x, edge_index, edge_type, batch, compress_W, compress_b, Wr1, Wroot1, b1, g1, be1, Wr2, Wroot2, b2, g2, be2, Wr3, Wroot3, b3, g3, be3, Wc, bc):
    src, dst = edge_index[0], edge_index[1]
    pad = EP - E
    src2 = jnp.pad(src, (0, pad)).reshape(EROWS, 128)
    dst2 = jnp.pad(dst, (0, pad), constant_values=N).reshape(EROWS, 128)
    et2 = jnp.pad(edge_type, (0, pad)).reshape(EROWS, 128)
    gidx2, key2 = _eprep(src2, dst2, et2)
    norm2 = _scprep(key2)
    h = _pre(x, compress_W, compress_b[None, :])

    def layer(xin, d, Wr, Wroot, b, g, be):
        ha, hb = _einsum(xin, Wr, d)
        agga, aggb = _scagg(ha, hb, gidx2, dst2, norm2)
        z, st = _bnsum(agga, aggb, xin, Wroot, b[None, :], d)
        return _bnapply(z, st, g[None, :], be[None, :])

    h = layer(h, IN1, Wr1, Wroot1, b1, g1, be1)
    h = layer(h, OUT, Wr2, Wroot2, b2, g2, be2)
    h = layer(h, OUT, Wr3, Wroot3, b3, g3, be3)

    batch3 = batch.reshape(N, 1)
    wcp = jnp.pad(Wc, ((0, 0), (0, 125)))
    bcp = jnp.pad(bc, (0, 125))[None, :]
    out = _pool(h, batch3, wcp, bcp)
    return out[:, :3]


# speedup vs baseline: 5.7615x; 1.1481x over previous
import functools

import jax
import jax.numpy as jnp
from jax import lax
from jax.experimental import pallas as pl
from jax.experimental.pallas import tpu as pltpu
from jax.experimental.pallas import tpu_sc as plsc

N = 50000
E = 800000
R = 8
OUT = 64
EMB = 64
COMP = 32
FEAT = 128
IN1 = 96
G = 128

NC = 2
NS = 16

EP = 819200
EROWS = EP // 128
TROWS = EROWS // NS
HROWS = TROWS // 2
NPAIR = HROWS // 2
NKEY = 409600
SROWS = 20
NSLAB = TROWS // SROWS
SPAIR = SROWS // 2
NTAB = R * N
NAGG = 50048
NAGT = NAGG // NS
NB = 25
BLK = 2000

_MESH = plsc.VectorSubcoreMesh(
    core_axis_name="c", subcore_axis_name="s", num_cores=NC, num_subcores=NS)
_SC_PARAMS = pltpu.CompilerParams(use_tc_tiling_on_sc=False)


def _pre_body(x_ref, w_ref, b_ref, o_ref):
    xb = x_ref[...]
    emb = xb[:, :EMB] @ w_ref[...] + b_ref[...]
    o_ref[...] = jnp.concatenate([emb, xb[:, EMB:]], axis=1)


def _pre(x, w, b):
    return pl.pallas_call(
        _pre_body,
        grid=(NB,),
        in_specs=[
            pl.BlockSpec((BLK, FEAT), lambda n: (n, 0)),
            pl.BlockSpec((EMB, COMP), lambda n: (0, 0)),
            pl.BlockSpec((1, COMP), lambda n: (0, 0)),
        ],
        out_specs=pl.BlockSpec((BLK, IN1), lambda n: (n, 0)),
        out_shape=jax.ShapeDtypeStruct((N, IN1), jnp.float32),
    )(x, w, b)


def _eprep_body(src_ref, dst_ref, et_ref, gidx_ref, key_ref):
    src = src_ref[...]
    dst = dst_ref[...]
    et = et_ref[...]
    gidx_ref[...] = et * N + src
    key_ref[...] = dst * R + et


def _eprep(src2, dst2, et2):
    return pl.pallas_call(
        _eprep_body,
        out_shape=[
            jax.ShapeDtypeStruct((EROWS, 128), jnp.int32),
            jax.ShapeDtypeStruct((EROWS, 128), jnp.int32),
        ],
    )(src2, dst2, et2)


def _einsum_body(x_ref, wr_ref, ha_ref, hb_ref):
    h = x_ref[...] @ wr_ref[0]
    ha_ref[...] = h[:, :32]
    hb_ref[...] = h[:, 32:]


def _einsum(x, wr, d):
    return pl.pallas_call(
        _einsum_body,
        grid=(R, NB),
        in_specs=[
            pl.BlockSpec((BLK, d), lambda r, n: (n, 0)),
            pl.BlockSpec((1, d, OUT), lambda r, n: (r, 0, 0)),
        ],
        out_specs=[
            pl.BlockSpec((BLK, 32), lambda r, n: (r * NB + n, 0)),
            pl.BlockSpec((BLK, 32), lambda r, n: (r * NB + n, 0)),
        ],
        out_shape=[
            jax.ShapeDtypeStruct((NTAB, 32), jnp.float32),
            jax.ShapeDtypeStruct((NTAB, 32), jnp.float32),
        ],
    )(x, wr)


def _bnsum_body(agga_ref, aggb_ref, x_ref, wr_ref, b_ref, z_ref, st_ref):
    agg = jnp.concatenate([agga_ref[...], aggb_ref[...]], axis=1)
    z = agg + x_ref[...] @ wr_ref[...] + b_ref[...]
    z_ref[...] = z
    s = jnp.sum(z, axis=0, keepdims=True)
    s2 = jnp.sum(z * z, axis=0, keepdims=True)
    st = jnp.concatenate([s, s2], axis=0)

    @pl.when(pl.program_id(0) == 0)
    def _():
        st_ref[...] = st

    @pl.when(pl.program_id(0) != 0)
    def _():
        st_ref[...] += st


def _bnsum(agga, aggb, x, wroot, b, d):
    return pl.pallas_call(
        _bnsum_body,
        grid=(NB,),
        in_specs=[
            pl.BlockSpec((BLK, 32), lambda n: (n, 0)),
            pl.BlockSpec((BLK, 32), lambda n: (n, 0)),
            pl.BlockSpec((BLK, d), lambda n: (n, 0)),
            pl.BlockSpec((d, OUT), lambda n: (0, 0)),
            pl.BlockSpec((1, OUT), lambda n: (0, 0)),
        ],
        out_specs=[
            pl.BlockSpec((BLK, OUT), lambda n: (n, 0)),
            pl.BlockSpec((2, OUT), lambda n: (0, 0)),
        ],
        out_shape=[
            jax.ShapeDtypeStruct((N, OUT), jnp.float32),
            jax.ShapeDtypeStruct((2, OUT), jnp.float32),
        ],
    )(agga, aggb, x, wroot, b)


def _bnapply_body(z_ref, st_ref, g_ref, be_ref, o_ref):
    m = st_ref[0:1, :] / N
    v = st_ref[1:2, :] / N - m * m
    zn = g_ref[...] * (z_ref[...] - m) / jnp.sqrt(v + 1e-5) + be_ref[...]
    o_ref[...] = jnp.maximum(zn, 0.0)


def _bnapply(z, st, g, be):
    return pl.pallas_call(
        _bnapply_body,
        grid=(NB,),
        in_specs=[
            pl.BlockSpec((BLK, OUT), lambda n: (n, 0)),
            pl.BlockSpec((2, OUT), lambda n: (0, 0)),
            pl.BlockSpec((1, OUT), lambda n: (0, 0)),
            pl.BlockSpec((1, OUT), lambda n: (0, 0)),
        ],
        out_specs=pl.BlockSpec((BLK, OUT), lambda n: (n, 0)),
        out_shape=jax.ShapeDtypeStruct((N, OUT), jnp.float32),
    )(z, st, g, be)


def _pool_body(h_ref, b_ref, wc_ref, bc_ref, o_ref, ssum, smax, scnt):
    n = pl.program_id(0)

    @pl.when(n == 0)
    def _():
        ssum[...] = jnp.zeros((G, OUT), jnp.float32)
        smax[...] = jnp.full((G, OUT), -1e30, jnp.float32)
        scnt[...] = jnp.zeros((G, 8), jnp.float32)

    hb = h_ref[...]
    bb = b_ref[...]
    lo = b_ref[0, 0]
    hi = b_ref[BLK - 1, 0]

    def body(g, _):
        mask = bb == g
        s_g = jnp.sum(jnp.where(mask, hb, 0.0), axis=0, keepdims=True)
        m_g = jnp.max(jnp.where(mask, hb, -1e30), axis=0, keepdims=True)
        c_g = jnp.sum(mask.astype(jnp.float32))
        ssum[pl.ds(g, 1), :] += s_g
        smax[pl.ds(g, 1), :] = jnp.maximum(smax[pl.ds(g, 1), :], m_g)
        scnt[pl.ds(g, 1), :] += c_g
        return 0

    lax.fori_loop(lo, hi + 1, body, 0)

    @pl.when(n == NB - 1)
    def _():
        cnt = scnt[:, 0:1]
        mean = ssum[...] / jnp.maximum(cnt, 1.0)
        mx = jnp.where(cnt > 0, smax[...], 0.0)
        pooled = jnp.concatenate([mean, mx], axis=1)
        o_ref[...] = pooled @ wc_ref[...] + bc_ref[...]


def _pool(h, batch3, wcp, bcp):
    return pl.pallas_call(
        _pool_body,
        grid=(NB,),
        in_specs=[
            pl.BlockSpec((BLK, OUT), lambda n: (n, 0)),
            pl.BlockSpec((BLK, 1), lambda n: (n, 0)),
            pl.BlockSpec((2 * OUT, 128), lambda n: (0, 0)),
            pl.BlockSpec((1, 128), lambda n: (0, 0)),
        ],
        out_specs=pl.BlockSpec((G, 128), lambda n: (0, 0)),
        out_shape=jax.ShapeDtypeStruct((G, 128), jnp.float32),
        scratch_shapes=[
            pltpu.VMEM((G, OUT), jnp.float32),
            pltpu.VMEM((G, OUT), jnp.float32),
            pltpu.VMEM((G, 8), jnp.float32),
        ],
    )(h, batch3, wcp, bcp)


@functools.partial(
    pl.kernel,
    out_type=jax.ShapeDtypeStruct((EROWS, 128), jnp.float32),
    mesh=_MESH,
    compiler_params=_SC_PARAMS,
    scratch_types=[
        pltpu.VMEM_SHARED((NKEY,), jnp.float32),
        pltpu.VMEM((TROWS, 128), jnp.int32),
        pltpu.VMEM((HROWS, 128), jnp.float32),
        pltpu.VMEM((128,), jnp.float32),
        pltpu.VMEM((128,), jnp.float32),
        pltpu.VMEM((1600,), jnp.float32),
    ],
)
def _scprep(key_hbm, norm_hbm, cnt_sh, kbuf, nbuf, ones_v, cval, zbuf):
    c = lax.axis_index("c")
    s = lax.axis_index("s")
    zero16 = jnp.zeros((16,), jnp.float32)
    one16 = jnp.ones((16,), jnp.float32)
    for i in range(100):
        zbuf[pl.ds(i * 16, 16)] = zero16
    for i in range(8):
        ones_v[pl.ds(i * 16, 16)] = one16
    for i in range(16):
        pltpu.sync_copy(zbuf, cnt_sh.at[pl.ds(s * 25600 + i * 1600, 1600)])
    plsc.subcore_barrier()
    pltpu.sync_copy(key_hbm.at[pl.ds(s * TROWS, TROWS)], kbuf)

    def count_chunk(i, _):
        pltpu.sync_copy(ones_v, cnt_sh.at[kbuf.at[i]], add=True)
        return 0

    lax.fori_loop(0, TROWS, count_chunk, 0)
    plsc.subcore_barrier()
    w = s * NC + c
    pltpu.sync_copy(key_hbm.at[pl.ds(w * HROWS, HROWS)],
                    kbuf.at[pl.ds(0, HROWS)])

    def norm_chunk(i, _):
        pltpu.sync_copy(cnt_sh.at[kbuf.at[i]], cval)
        for j in range(8):
            v = cval[pl.ds(j * 16, 16)]
            nbuf[i, pl.ds(j * 16, 16)] = 1.0 / jnp.maximum(v, 1.0)
        return 0

    lax.fori_loop(0, HROWS, norm_chunk, 0)
    pltpu.sync_copy(nbuf, norm_hbm.at[pl.ds(w * HROWS, HROWS)])


@functools.partial(
    pl.kernel,
    out_type=[
        jax.ShapeDtypeStruct((NAGG, 32), jnp.float32),
        jax.ShapeDtypeStruct((NAGG, 32), jnp.float32),
    ],
    mesh=_MESH,
    compiler_params=_SC_PARAMS,
    scratch_types=[
        pltpu.VMEM_SHARED((NAGG, 32), jnp.float32),
        pltpu.VMEM((SROWS, 128), jnp.int32),
        pltpu.VMEM((SROWS, 128), jnp.int32),
        pltpu.VMEM((SROWS, 128), jnp.float32),
        pltpu.VMEM((128, 32), jnp.float32),
        pltpu.VMEM((128, 32), jnp.float32),
        pltpu.VMEM((128, 32), jnp.float32),
        pltpu.VMEM((128, 32), jnp.float32),
        pltpu.SemaphoreType.DMA,
        pltpu.SemaphoreType.DMA,
        pltpu.SemaphoreType.DMA,
        pltpu.SemaphoreType.DMA,
    ],
)
def _scagg(ha_hbm, hb_hbm, gidx_hbm, dst_hbm, norm_hbm, outa_hbm, outb_hbm,
           agg_sh, gbuf, dbuf, nbuf, rg0, rg1, rs0, rs1,
           gsem0, gsem1, ssem0, ssem1):
    c = lax.axis_index("c")
    s = lax.axis_index("s")
    zero16 = jnp.zeros((16,), jnp.float32)
    for r in range(128):
        rg0[r, pl.ds(0, 16)] = zero16
        rg0[r, pl.ds(16, 16)] = zero16
    for i in range(24):
        pltpu.sync_copy(rg0, agg_sh.at[pl.ds(s * NAGT + i * 128, 128)])
    pltpu.sync_copy(rg0.at[pl.ds(0, 56)],
                    agg_sh.at[pl.ds(s * NAGT + 24 * 128, 56)])
    plsc.subcore_barrier()

    splats = [jnp.full((16,), l, jnp.int32) for l in range(16)]

    def scale(src, dst, i):
        for j in range(8):
            nv = nbuf[i, pl.ds(j * 16, 16)]
            for l in range(16):
                e = j * 16 + l
                sc = nv.at[splats[l]].get(mode='promise_in_bounds')
                dst[e, pl.ds(0, 16)] = src[e, pl.ds(0, 16)] * sc
                dst[e, pl.ds(16, 16)] = src[e, pl.ds(16, 16)] * sc

    def run(table):
        def slab(si, _):
            base = s * TROWS + si * SROWS
            pltpu.sync_copy(gidx_hbm.at[pl.ds(base, SROWS)], gbuf)
            pltpu.sync_copy(dst_hbm.at[pl.ds(base, SROWS)], dbuf)
            pltpu.sync_copy(norm_hbm.at[pl.ds(base, SROWS)], nbuf)
            pltpu.async_copy(table.at[gbuf.at[0]], rg0, gsem0)
            pltpu.async_copy(table.at[gbuf.at[1]], rg1, gsem1)

            def pair(j, _):
                a = 2 * j
                b = a + 1
                pltpu.make_async_copy(table.at[gbuf.at[a]], rg0, gsem0).wait()

                @pl.when(j > 0)
                def _():
                    pltpu.make_async_copy(
                        rs0, agg_sh.at[dbuf.at[a]], ssem0).wait()

                scale(rg0, rs0, a)

                @pl.when(j < SPAIR - 1)
                def _():
                    pltpu.async_copy(table.at[gbuf.at[a + 2]], rg0, gsem0)

                pltpu.async_copy(rs0, agg_sh.at[dbuf.at[a]], ssem0, add=True)
                pltpu.make_async_copy(table.at[gbuf.at[b]], rg1, gsem1).wait()

                @pl.when(j > 0)
                def _():
                    pltpu.make_async_copy(
                        rs1, agg_sh.at[dbuf.at[b]], ssem1).wait()

                scale(rg1, rs1, b)

                @pl.when(j < SPAIR - 1)
                def _():
                    pltpu.async_copy(table.at[gbuf.at[b + 2]], rg1, gsem1)

                pltpu.async_copy(rs1, agg_sh.at[dbuf.at[b]], ssem1, add=True)
                return 0

            lax.fori_loop(0, SPAIR, pair, 0)
            pltpu.make_async_copy(
                rs0, agg_sh.at[dbuf.at[SROWS - 2]], ssem0).wait()
            pltpu.make_async_copy(
                rs1, agg_sh.at[dbuf.at[SROWS - 1]], ssem1).wait()
            return 0

        lax.fori_loop(0, NSLAB, slab, 0)

    @pl.when(c == 0)
    def _():
        run(ha_hbm)

    @pl.when(c == 1)
    def _():
        run(hb_hbm)

    plsc.subcore_barrier()

    @pl.when(c == 0)
    def _():
        pltpu.sync_copy(agg_sh.at[pl.ds(s * NAGT, NAGT)],
                        outa_hbm.at[pl.ds(s * NAGT, NAGT)])

    @pl.when(c == 1)
    def _():
        pltpu.sync_copy(agg_sh.at[pl.ds(s * NAGT, NAGT)],
                        outb_hbm.at[pl.ds(s * NAGT, NAGT)])


def kernel(x, edge_index, edge_type, batch, compress_W, compress_b, Wr1, Wroot1, b1, g1, be1, Wr2, Wroot2, b2, g2, be2, Wr3, Wroot3, b3, g3, be3, Wc, bc):
    src, dst = edge_index[0], edge_index[1]
    pad = EP - E
    src2 = jnp.pad(src, (0, pad)).reshape(EROWS, 128)
    dst2 = jnp.pad(dst, (0, pad), constant_values=N).reshape(EROWS, 128)
    et2 = jnp.pad(edge_type, (0, pad)).reshape(EROWS, 128)
    gidx2, key2 = _eprep(src2, dst2, et2)
    norm2 = _scprep(key2)
    h = _pre(x, compress_W, compress_b[None, :])

    def layer(xin, d, Wr, Wroot, b, g, be):
        ha, hb = _einsum(xin, Wr, d)
        agga, aggb = _scagg(ha, hb, gidx2, dst2, norm2)
        z, st = _bnsum(agga, aggb, xin, Wroot, b[None, :], d)
        return _bnapply(z, st, g[None, :], be[None, :])

    h = layer(h, IN1, Wr1, Wroot1, b1, g1, be1)
    h = layer(h, OUT, Wr2, Wroot2, b2, g2, be2)
    h = layer(h, OUT, Wr3, Wroot3, b3, g3, be3)

    batch3 = batch.reshape(N, 1)
    wcp = jnp.pad(Wc, ((0, 0), (0, 125)))
    bcp = jnp.pad(bc, (0, 125))[None, :]
    out = _pool(h, batch3, wcp, bcp)
    return out[:, :3]
